# baseline (speedup 1.0000x reference)
import functools

import jax
import jax.numpy as jnp
from jax import lax
from jax.experimental import pallas as pl
from jax.experimental.pallas import tpu as pltpu, tpu_sc as plsc

_BATCH = 16384
_DIM = 64


@functools.partial(jax.jit, static_argnames=())
def _lookup(die_idx, die_embedding):
    info = plsc.get_sparse_core_info()
    nw = info.num_cores * info.num_subcores
    b_per_w = _BATCH // nw

    mesh = plsc.VectorSubcoreMesh(core_axis_name="c", subcore_axis_name="s")

    @functools.partial(
        pl.kernel,
        mesh=mesh,
        compiler_params=pltpu.CompilerParams(use_tc_tiling_on_sc=False),
        out_type=jax.ShapeDtypeStruct((_BATCH, _DIM), jnp.float32),
        scratch_types=[
            pltpu.VMEM((b_per_w,), jnp.int32),
            pltpu.VMEM((b_per_w, _DIM), jnp.float32),
            pltpu.SemaphoreType.DMA,
        ],
    )
    def k(idx_hbm, table_hbm, out_hbm, idx_v, rows_v, sem):
        wid = lax.axis_index("s") * info.num_cores + lax.axis_index("c")
        base = wid * b_per_w
        pltpu.sync_copy(idx_hbm.at[pl.ds(base, b_per_w)], idx_v)
        pltpu.async_copy(table_hbm.at[idx_v], rows_v, sem).wait()
        pltpu.sync_copy(rows_v, out_hbm.at[pl.ds(base, b_per_w)])

    return k(die_idx, die_embedding)


def kernel(die_idx, die_embedding):
    return _lookup(die_idx.astype(jnp.int32), die_embedding)


# speedup vs baseline: 1.4958x; 1.4958x over previous
import functools

import jax
import jax.numpy as jnp
from jax import lax
from jax.experimental import pallas as pl
from jax.experimental.pallas import tpu as pltpu, tpu_sc as plsc

_BATCH = 16384
_DIM = 64


@jax.jit
def _lookup(die_idx, die_embedding):
    info = plsc.get_sparse_core_info()
    nw = info.num_cores * info.num_subcores
    b_per_w = _BATCH // nw

    mesh = plsc.VectorSubcoreMesh(core_axis_name="c", subcore_axis_name="s")

    @functools.partial(
        pl.kernel,
        mesh=mesh,
        out_type=jax.ShapeDtypeStruct((_BATCH, _DIM), jnp.float32),
        scratch_types=[
            pltpu.VMEM((b_per_w,), jnp.int32),
            pltpu.VMEM((b_per_w, _DIM), jnp.float32),
            pltpu.SemaphoreType.DMA,
        ],
    )
    def k(idx_hbm, table_hbm, out_hbm, idx_v, rows_v, sem):
        wid = lax.axis_index("s") * info.num_cores + lax.axis_index("c")
        base = wid * b_per_w
        pltpu.sync_copy(idx_hbm.at[pl.ds(base, b_per_w)], idx_v)

        def fire_group(g, carry):
            vec = idx_v[pl.ds(g * info.num_lanes, info.num_lanes)]
            for l in range(info.num_lanes):
                i = vec[l]
                pltpu.make_async_copy(
                    table_hbm.at[pl.ds(i, 1)],
                    rows_v.at[pl.ds(g * info.num_lanes + l, 1)],
                    sem,
                ).start()
            return carry

        lax.fori_loop(0, b_per_w // info.num_lanes, fire_group, 0)
        pltpu.make_async_copy(
            table_hbm.at[pl.ds(0, b_per_w)], rows_v, sem
        ).wait()
        pltpu.sync_copy(rows_v, out_hbm.at[pl.ds(base, b_per_w)])

    return k(die_idx, die_embedding)


def kernel(die_idx, die_embedding):
    return _lookup(die_idx.astype(jnp.int32), die_embedding)


# speedup vs baseline: 1.8908x; 1.2641x over previous
import functools

import jax
import jax.numpy as jnp
from jax import lax
from jax.experimental import pallas as pl
from jax.experimental.pallas import tpu as pltpu, tpu_sc as plsc

_BATCH = 16384
_DIM = 64
_ROWS = 100001
_ROW_PAD = 100008
_CHUNK = 8192


@jax.jit
def _lookup(die_idx, table_t):
    info = plsc.get_sparse_core_info()
    nw = info.num_cores * info.num_subcores
    nl = info.num_lanes
    rows_per_w = _DIM // nw

    mesh = plsc.VectorSubcoreMesh(core_axis_name="c", subcore_axis_name="s")

    @functools.partial(
        pl.kernel,
        mesh=mesh,
        compiler_params=pltpu.CompilerParams(needs_layout_passes=False),
        out_type=jax.ShapeDtypeStruct((_DIM, _BATCH), jnp.float32),
        scratch_types=[
            pltpu.VMEM((1, _ROWS), jnp.float32),
            pltpu.VMEM((_BATCH,), jnp.int32),
            pltpu.VMEM((1, _CHUNK), jnp.float32),
        ],
    )
    def k(idx_hbm, table_hbm, out_hbm, row_v, idx_v, outc_v):
        wid = lax.axis_index("s") * info.num_cores + lax.axis_index("c")
        pltpu.sync_copy(idx_hbm, idx_v)
        zv = jnp.zeros((nl,), jnp.int32)
        for row in range(rows_per_w):
            r = wid * rows_per_w + row
            pltpu.sync_copy(table_hbm.at[pl.ds(r, 1), :], row_v)
            for chunk in range(_BATCH // _CHUNK):
                def gather_group(g, carry, chunk=chunk):
                    base = chunk * _CHUNK + g * nl
                    iv = idx_v[pl.ds(base, nl)]
                    outc_v[0, pl.ds(g * nl, nl)] = plsc.load_gather(
                        row_v, [zv, iv]
                    )
                    return carry

                lax.fori_loop(0, _CHUNK // nl, gather_group, 0, unroll=8)
                pltpu.sync_copy(
                    outc_v,
                    out_hbm.at[pl.ds(r, 1), pl.ds(chunk * _CHUNK, _CHUNK)],
                )

    return k(die_idx, table_t)


def kernel(die_idx, die_embedding):
    out_t = _lookup(die_idx.astype(jnp.int32), die_embedding.T)
    return out_t.T
